# baseline (speedup 1.0000x reference)
import functools

import jax
import jax.numpy as jnp
from jax import lax
from jax.experimental import pallas as pl
from jax.experimental.pallas import tpu as pltpu
from jax.experimental.pallas import tpu_sc as plsc

B = 16384
D = 16

_NC = 2
_NS = 16
_NW = _NC * _NS
_BPW = B // _NW


def _sc_gather_mul(user_ids, item_ids, ut_T, it_T):
    mesh = plsc.VectorSubcoreMesh(core_axis_name="c", subcore_axis_name="s")

    @functools.partial(
        pl.kernel,
        out_type=jax.ShapeDtypeStruct((D, B), jnp.float32),
        mesh=mesh,
        compiler_params=pltpu.CompilerParams(use_tc_tiling_on_sc=False),
        scratch_types=[
            pltpu.VMEM((_BPW,), jnp.int32),
            pltpu.VMEM((_BPW,), jnp.int32),
            pltpu.VMEM((D, _BPW), jnp.float32),
            pltpu.VMEM((D, _BPW), jnp.float32),
            pltpu.SemaphoreType.DMA,
            pltpu.SemaphoreType.DMA,
        ],
    )
    def body(uids_hbm, iids_hbm, ut_hbm, it_hbm, out_hbm,
             uidx_v, iidx_v, urows_v, irows_v, usem, isem):
        wid = lax.axis_index("s") * _NC + lax.axis_index("c")
        base = wid * _BPW
        pltpu.sync_copy(uids_hbm.at[pl.ds(base, _BPW)], uidx_v)
        pltpu.sync_copy(iids_hbm.at[pl.ds(base, _BPW)], iidx_v)
        ucopies = []
        icopies = []
        for d in range(D):
            ucopies.append(
                pltpu.async_copy(ut_hbm.at[d].at[uidx_v], urows_v.at[d], usem))
            icopies.append(
                pltpu.async_copy(it_hbm.at[d].at[iidx_v], irows_v.at[d], isem))
        for c in ucopies:
            c.wait()
        for c in icopies:
            c.wait()

        def mul_chunk(c, carry):
            for d in range(D):
                urows_v[d, pl.ds(c * 16, 16)] = (
                    urows_v[d, pl.ds(c * 16, 16)] * irows_v[d, pl.ds(c * 16, 16)])
            return carry

        lax.fori_loop(0, _BPW // 16, mul_chunk, 0)
        pltpu.sync_copy(urows_v, out_hbm.at[:, pl.ds(base, _BPW)])

    return body(user_ids, item_ids, ut_T, it_T)


_BLK = 2048


def _mlp_body(x_ref, w1_ref, b1_ref, w2_ref, b2_ref, w3_ref, b3_ref, o_ref):
    xt = x_ref[...]
    h = lax.dot_general(w1_ref[...], xt, (((0,), (0,)), ((), ())),
                        preferred_element_type=jnp.float32)
    h = jnp.maximum(h + b1_ref[...], 0.0)
    h = lax.dot_general(w2_ref[...], h, (((0,), (0,)), ((), ())),
                        preferred_element_type=jnp.float32)
    h = jnp.maximum(h + b2_ref[...], 0.0)
    o_ref[...] = lax.dot_general(w3_ref[...], h, (((0,), (0,)), ((), ())),
                                 preferred_element_type=jnp.float32) + b3_ref[...]


def _tc_mlp(xt, W1, b1, W2, b2, W3, b3):
    grid = (B // _BLK,)
    return pl.pallas_call(
        _mlp_body,
        grid=grid,
        in_specs=[
            pl.BlockSpec((D, _BLK), lambda i: (0, i)),
            pl.BlockSpec((D, 128), lambda i: (0, 0)),
            pl.BlockSpec((128, 1), lambda i: (0, 0)),
            pl.BlockSpec((128, 64), lambda i: (0, 0)),
            pl.BlockSpec((64, 1), lambda i: (0, 0)),
            pl.BlockSpec((64, 1), lambda i: (0, 0)),
            pl.BlockSpec((1, 1), lambda i: (0, 0)),
        ],
        out_specs=pl.BlockSpec((1, _BLK), lambda i: (0, i)),
        out_shape=jax.ShapeDtypeStruct((1, B), jnp.float32),
    )(xt, W1, b1.reshape(128, 1), W2, b2.reshape(64, 1), W3, b3.reshape(1, 1))


def kernel(user_ids, item_ids, user_table, item_table, W1, b1, W2, b2, W3, b3):
    xt = _sc_gather_mul(user_ids.astype(jnp.int32), item_ids.astype(jnp.int32),
                        user_table.T, item_table.T)
    out = _tc_mlp(xt, W1, b1, W2, b2, W3, b3)
    return out.reshape(B)


# speedup vs baseline: 3.1102x; 3.1102x over previous
import functools

import jax
import jax.numpy as jnp
from jax import lax
from jax.experimental import pallas as pl
from jax.experimental.pallas import tpu as pltpu
from jax.experimental.pallas import tpu_sc as plsc

B = 16384
D = 16

_NC = 2
_NS = 16
_NW = _NC * _NS
_BPW = B // _NW


def _sc_gather_mul(user_ids, item_ids, user_table, item_table):
    mesh = plsc.VectorSubcoreMesh(core_axis_name="c", subcore_axis_name="s")

    @functools.partial(
        pl.kernel,
        out_type=jax.ShapeDtypeStruct((B, D), jnp.float32),
        mesh=mesh,
        compiler_params=pltpu.CompilerParams(use_tc_tiling_on_sc=False),
        scratch_types=[
            pltpu.VMEM((_BPW,), jnp.int32),
            pltpu.VMEM((_BPW,), jnp.int32),
            pltpu.VMEM((_BPW, D), jnp.float32),
            pltpu.VMEM((_BPW, D), jnp.float32),
            pltpu.SemaphoreType.DMA,
            pltpu.SemaphoreType.DMA,
        ],
    )
    def body(uids_hbm, iids_hbm, ut_hbm, it_hbm, out_hbm,
             uidx_v, iidx_v, urows_v, irows_v, usem, isem):
        wid = lax.axis_index("s") * _NC + lax.axis_index("c")
        base = wid * _BPW
        pltpu.sync_copy(uids_hbm.at[pl.ds(base, _BPW)], uidx_v)
        pltpu.sync_copy(iids_hbm.at[pl.ds(base, _BPW)], iidx_v)
        cu = pltpu.async_copy(ut_hbm.at[uidx_v], urows_v, usem)
        ci = pltpu.async_copy(it_hbm.at[iidx_v], irows_v, isem)
        cu.wait()
        ci.wait()

        def mul_row(r, carry):
            urows_v[r] = urows_v[r] * irows_v[r]
            return carry

        lax.fori_loop(0, _BPW, mul_row, 0)
        pltpu.sync_copy(urows_v, out_hbm.at[pl.ds(base, _BPW)])

    return body(user_ids, item_ids, user_table, item_table)


_BLK = 2048


def _mlp_body(x_ref, w1_ref, b1_ref, w2_ref, b2_ref, w3_ref, b3_ref, o_ref):
    h = jnp.dot(x_ref[...], w1_ref[...], preferred_element_type=jnp.float32)
    h = jnp.maximum(h + b1_ref[...], 0.0)
    h = jnp.dot(h, w2_ref[...], preferred_element_type=jnp.float32)
    h = jnp.maximum(h + b2_ref[...], 0.0)
    o_ref[...] = jnp.dot(h, w3_ref[...], preferred_element_type=jnp.float32) + b3_ref[...]


def _tc_mlp(x, W1, b1, W2, b2, W3, b3):
    grid = (B // _BLK,)
    return pl.pallas_call(
        _mlp_body,
        grid=grid,
        in_specs=[
            pl.BlockSpec((_BLK, D), lambda i: (i, 0)),
            pl.BlockSpec((D, 128), lambda i: (0, 0)),
            pl.BlockSpec((1, 128), lambda i: (0, 0)),
            pl.BlockSpec((128, 64), lambda i: (0, 0)),
            pl.BlockSpec((1, 64), lambda i: (0, 0)),
            pl.BlockSpec((64, 1), lambda i: (0, 0)),
            pl.BlockSpec((1, 1), lambda i: (0, 0)),
        ],
        out_specs=pl.BlockSpec((_BLK, 1), lambda i: (i, 0)),
        out_shape=jax.ShapeDtypeStruct((B, 1), jnp.float32),
    )(x, W1, b1.reshape(1, 128), W2, b2.reshape(1, 64), W3, b3.reshape(1, 1))


def kernel(user_ids, item_ids, user_table, item_table, W1, b1, W2, b2, W3, b3):
    x = _sc_gather_mul(user_ids.astype(jnp.int32), item_ids.astype(jnp.int32),
                       user_table, item_table)
    out = _tc_mlp(x, W1, b1, W2, b2, W3, b3)
    return out.reshape(B)
